# Initial kernel scaffold; baseline (speedup 1.0000x reference)
import jax
import jax.numpy as jnp
from jax.experimental import pallas as pl


def kernel(nodes, to_neighs, u2e, W1, b1, W2, b2, W3, b3):
    raise NotImplementedError("write your pallas kernel here")


# speedup vs baseline: 2.2256x; 2.2256x over previous
import functools

import jax
import jax.numpy as jnp
from jax import lax
from jax.experimental import pallas as pl
from jax.experimental.pallas import tpu as pltpu
from jax.experimental.pallas import tpu_sc as plsc

N_USERS = 100000
D = 128
N_NODES = 10000
DEG = 32

NW = 32
CHUNK = 128

_RAW = N_NODES * DEG + N_NODES
_TOT = ((_RAW + NW * CHUNK - 1) // (NW * CHUNK)) * (NW * CHUNK)
_PER_W = _TOT // NW
_N_CHUNK = _PER_W // CHUNK

BN = 200
GRID = N_NODES // BN


def _sc_gather(idx, table):
    mesh = plsc.VectorSubcoreMesh(core_axis_name="c", subcore_axis_name="s")

    @functools.partial(
        pl.kernel,
        out_type=jax.ShapeDtypeStruct((_TOT, D), jnp.float32),
        mesh=mesh,
        scratch_types=[
            pltpu.VMEM((CHUNK,), jnp.int32),
            pltpu.VMEM((CHUNK, D), jnp.float32),
            pltpu.SemaphoreType.DMA,
        ],
    )
    def gather_kernel(idx_hbm, table_hbm, out_hbm, idx_v, rows_v, sem):
        nc = 2
        wid = lax.axis_index("s") * nc + lax.axis_index("c")
        base = wid * _PER_W

        def body(c, _):
            off = base + c * CHUNK
            pltpu.sync_copy(idx_hbm.at[pl.ds(off, CHUNK)], idx_v)
            pltpu.async_copy(table_hbm.at[idx_v], rows_v, sem).wait()
            pltpu.sync_copy(rows_v, out_hbm.at[pl.ds(off, CHUNK)])
            return ()

        lax.fori_loop(0, _N_CHUNK, body, ())

    return gather_kernel(idx, table)


def _tc_body(eu_ref, u_ref, w1a_ref, w1b_ref, b1_ref, w2_ref, b2_ref,
             w3_ref, out_ref):
    eu = eu_ref[...]
    u = u_ref[...]
    u_part = jnp.dot(u, w1b_ref[...],
                     preferred_element_type=jnp.float32) + b1_ref[...]
    h = jnp.dot(eu, w1a_ref[...], preferred_element_type=jnp.float32)
    h = h.reshape(BN, DEG, D) + u_part[:, None, :]
    h = jnp.maximum(h, 0.0).reshape(BN * DEG, D)
    h = jnp.dot(h, w2_ref[...], preferred_element_type=jnp.float32)
    h = jnp.maximum(h + b2_ref[...], 0.0)
    logits = jnp.sum(h * w3_ref[...], axis=1).reshape(BN, DEG)
    logits = logits - jnp.max(logits, axis=1, keepdims=True)
    e = jnp.exp(logits)
    att = e / jnp.sum(e, axis=1, keepdims=True)
    w = att[:, :, None] * eu.reshape(BN, DEG, D)
    out_ref[...] = jnp.sum(w, axis=1)


def kernel(nodes, to_neighs, u2e, W1, b1, W2, b2, W3, b3):
    idx = jnp.concatenate([
        to_neighs.reshape(-1),
        nodes,
        jnp.zeros((_TOT - _RAW,), jnp.int32),
    ])
    rows = _sc_gather(idx, u2e)

    w1a = W1[:D]
    w1b = W1[D:]
    b1r = b1.reshape(1, D)
    b2r = b2.reshape(1, D)
    w3r = W3.reshape(1, D)

    grid_spec = pl.GridSpec(
        grid=(GRID,),
        in_specs=[
            pl.BlockSpec((BN * DEG, D), lambda i: (i, 0)),
            pl.BlockSpec((BN, D), lambda i: (N_NODES * DEG // BN + i, 0)),
            pl.BlockSpec((D, D), lambda i: (0, 0)),
            pl.BlockSpec((D, D), lambda i: (0, 0)),
            pl.BlockSpec((1, D), lambda i: (0, 0)),
            pl.BlockSpec((D, D), lambda i: (0, 0)),
            pl.BlockSpec((1, D), lambda i: (0, 0)),
            pl.BlockSpec((1, D), lambda i: (0, 0)),
        ],
        out_specs=pl.BlockSpec((BN, D), lambda i: (i, 0)),
    )
    return pl.pallas_call(
        _tc_body,
        grid_spec=grid_spec,
        out_shape=jax.ShapeDtypeStruct((N_NODES, D), jnp.float32),
        compiler_params=pltpu.CompilerParams(
            dimension_semantics=("arbitrary",),
        ),
    )(rows, rows, w1a, w1b, b1r, W2, b2r, w3r)
